# baseline (speedup 1.0000x reference)
import functools

import jax
import jax.numpy as jnp
from jax import lax
from jax.experimental import pallas as pl
from jax.experimental.pallas import tpu as pltpu
from jax.experimental.pallas import tpu_sc as plsc

_T, _VOCAB, _DIM, _B = 26, 100000, 128, 4096
_N = _T * _B
_NW = 32
_CH = 128
_NB = 4
_NT = _T // 2
_NH = _NT * _B


def _sc_gather(indices, qpack, sflat, bflat):
  nh = indices.shape[0]
  pw = nh // _NW
  nch = pw // _CH
  mesh = plsc.VectorSubcoreMesh(core_axis_name="c", subcore_axis_name="s")

  @functools.partial(
      pl.kernel,
      mesh=mesh,
      compiler_params=pltpu.CompilerParams(use_tc_tiling_on_sc=False),
      out_type=[
          jax.ShapeDtypeStruct((nh, _DIM), jnp.int32),
          jax.ShapeDtypeStruct((nh,), jnp.float32),
          jax.ShapeDtypeStruct((nh,), jnp.float32),
      ],
      scratch_types=[
          pltpu.VMEM((pw,), jnp.int32),
          pltpu.VMEM((nch, _CH), jnp.int32),
          pltpu.VMEM((nch, _CH), jnp.int32),
          pltpu.VMEM((_NB, _CH, _DIM), jnp.int32),
          pltpu.VMEM((pw,), jnp.float32),
          pltpu.VMEM((pw,), jnp.float32),
          pltpu.SemaphoreType.DMA,
          pltpu.SemaphoreType.DMA,
          pltpu.SemaphoreType.DMA,
      ],
  )
  def k(idx_hbm, q_hbm, s_hbm, b_hbm, qg_hbm, sg_hbm, bg_hbm,
        idx_raw, idx_v, idx_g, gran_v, s_v, b_v, sem_g, sem_o, sem_sb):
    wid = lax.axis_index("s") * 2 + lax.axis_index("c")
    base = wid * pw

    pltpu.sync_copy(idx_hbm.at[pl.ds(base, pw)], idx_raw)

    for c in range(nch):
      t_c = lax.shift_right_logical(base + c * _CH, 12)
      off = t_c * _VOCAB
      for j in range(_CH // 16):
        v = idx_raw[pl.ds(c * _CH + j * 16, 16)] + off
        idx_v[c, pl.ds(j * 16, 16)] = v
        idx_g[c, pl.ds(j * 16, 16)] = lax.shift_right_logical(v, 2)

    sb_pending = []
    for c in range(nch):
      sl = pl.ds(c * _CH, _CH)
      rows = idx_v.at[c]
      sb_pending.append(pltpu.async_copy(s_hbm.at[rows], s_v.at[sl], sem_sb))
      sb_pending.append(pltpu.async_copy(b_hbm.at[rows], b_v.at[sl], sem_sb))
      while len(sb_pending) > 8:
        sb_pending.pop(0).wait()

    g_h = [None] * _NB
    o_h = [None] * _NB
    for c in range(nch):
      b = c % _NB
      if o_h[b] is not None:
        o_h[b].wait()
      g_h[b] = pltpu.async_copy(q_hbm.at[idx_g.at[c]], gran_v.at[b], sem_g)
      if c >= _NB - 1:
        cd = c - (_NB - 1)
        bd = cd % _NB
        g_h[bd].wait()
        o_h[bd] = pltpu.async_copy(
            gran_v.at[bd], qg_hbm.at[pl.ds(base + cd * _CH, _CH)], sem_o)
    for cd in range(nch - _NB + 1, nch):
      bd = cd % _NB
      g_h[bd].wait()
      o_h[bd] = pltpu.async_copy(
          gran_v.at[bd], qg_hbm.at[pl.ds(base + cd * _CH, _CH)], sem_o)
    for h in o_h:
      if h is not None:
        h.wait()
    while sb_pending:
      sb_pending.pop(0).wait()

    pltpu.sync_copy(s_v, sg_hbm.at[pl.ds(base, pw)])
    pltpu.sync_copy(b_v, bg_hbm.at[pl.ds(base, pw)])

  return k(indices, qpack, sflat, bflat)


def _tc_repack(qflat):
  g = qflat.shape[0] // 4
  rr = 2600

  def body(q_ref, o_ref):
    o_ref[...] = pltpu.bitcast(q_ref[...], jnp.int32)

  return pl.pallas_call(
      body,
      grid=(g // rr,),
      in_specs=[pl.BlockSpec((4 * rr, _DIM), lambda i: (i, 0))],
      out_specs=pl.BlockSpec((rr, _DIM), lambda i: (i, 0)),
      out_shape=jax.ShapeDtypeStruct((g, _DIM), jnp.int32),
  )(qflat)


def _tc_dequant(qg, kshift, sg, bg):
  nt = qg.shape[0]
  bb = 1024

  def body(q_ref, k_ref, s_ref, b_ref, o_ref):
    q = lax.shift_right_logical(q_ref[0], k_ref[0]) & 0xFF
    x = q.astype(jnp.float32) * s_ref[0] + b_ref[0]
    bits = lax.bitcast_convert_type(x, jnp.int32)
    sgn = lax.shift_right_logical(bits, 16) & 0x8000
    mag = bits & 0x7FFFFFFF
    rnd = mag + 0xFFF + (lax.shift_right_logical(mag, 13) & 1)
    h = lax.shift_right_logical(rnd, 13) - 0x1C000
    h = jnp.where(mag < 0x38800000, 0, h)
    o_ref[...] = (sgn | h).astype(jnp.uint16)

  return pl.pallas_call(
      body,
      grid=(nt, _B // bb),
      in_specs=[
          pl.BlockSpec((1, bb, _DIM), lambda t, b: (t, b, 0)),
          pl.BlockSpec((1, bb, 1), lambda t, b: (t, b, 0)),
          pl.BlockSpec((1, bb, 1), lambda t, b: (t, b, 0)),
          pl.BlockSpec((1, bb, 1), lambda t, b: (t, b, 0)),
      ],
      out_specs=pl.BlockSpec((bb, _DIM), lambda t, b: (b, t)),
      out_shape=jax.ShapeDtypeStruct((_B, nt * _DIM), jnp.uint16),
  )(qg, kshift, sg, bg)


def kernel(indices, offsets, qweights, scales, biases):
  del offsets
  qflat = qweights.reshape(_T * _VOCAB, _DIM)
  sflat = scales.reshape(_T * _VOCAB)
  bflat = biases.reshape(_T * _VOCAB)
  kshift = ((indices & 3) * 8).reshape(_T, _B, 1)

  outs = []
  for c in range(_T // _NT):
    rsl = slice(c * _NT * _VOCAB, (c + 1) * _NT * _VOCAB)
    qpack = _tc_repack(qflat[rsl])
    qg, sg, bg = _sc_gather(
        indices[c * _NH:(c + 1) * _NH], qpack, sflat[rsl], bflat[rsl])
    outs.append(_tc_dequant(
        qg.reshape(_NT, _B, _DIM),
        kshift[c * _NT:(c + 1) * _NT],
        sg.reshape(_NT, _B, 1),
        bg.reshape(_NT, _B, 1),
    ))
  return lax.bitcast_convert_type(jnp.concatenate(outs, axis=1), jnp.float16)
